# baseline (speedup 1.0000x reference)
import functools

import jax
import jax.numpy as jnp
from jax import lax
from jax.experimental import pallas as pl
from jax.experimental.pallas import tpu as pltpu
from jax.experimental.pallas import tpu_sc as plsc

B = 1024
D = 128
C = 100000
P = 4
T = 1.0
P_MARGIN = 0.2
N_MARGIN = 0.3

BC = 3584
NB = (C + BC - 1) // BC
SACC = 256

_NC = 2
_NS = 16
_NW = _NC * _NS
_BP = P * B // _NW
_BT = B // _NW


@functools.partial(
    pl.kernel,
    mesh=plsc.VectorSubcoreMesh(core_axis_name="c", subcore_axis_name="s"),
    out_type=[
        jax.ShapeDtypeStruct((P * B, D), jnp.float32),
        jax.ShapeDtypeStruct((P * B, D), jnp.float32),
        jax.ShapeDtypeStruct((B, D), jnp.float32),
    ],
    scratch_types=[
        pltpu.VMEM((_BP,), jnp.int32),
        pltpu.VMEM((_BP,), jnp.int32),
        pltpu.VMEM((_BT,), jnp.int32),
        pltpu.VMEM((_BP, D), jnp.float32),
        pltpu.VMEM((_BP, D), jnp.float32),
        pltpu.VMEM((_BT, D), jnp.float32),
        pltpu.SemaphoreType.DMA,
        pltpu.SemaphoreType.DMA,
    ],
)
def _sc_gather(x_hbm, v_hbm, pidx_hbm, nidx_hbm, tgt_hbm,
               pg_hbm, ng_hbm, vt_hbm,
               pidx_v, nidx_v, tidx_v, prows_v, nrows_v, trows_v,
               sem, osem):
    wid = lax.axis_index("s") * _NC + lax.axis_index("c")
    base = wid * _BP
    tbase = wid * _BT
    i1 = pltpu.async_copy(pidx_hbm.at[pl.ds(base, _BP)], pidx_v, sem)
    i2 = pltpu.async_copy(nidx_hbm.at[pl.ds(base, _BP)], nidx_v, sem)
    i3 = pltpu.async_copy(tgt_hbm.at[pl.ds(tbase, _BT)], tidx_v, sem)
    i1.wait()
    g1 = pltpu.async_copy(x_hbm.at[pidx_v], prows_v, sem)
    i2.wait()
    g2 = pltpu.async_copy(x_hbm.at[nidx_v], nrows_v, sem)
    i3.wait()
    g3 = pltpu.async_copy(v_hbm.at[tidx_v], trows_v, sem)
    g1.wait()
    o1 = pltpu.async_copy(prows_v, pg_hbm.at[pl.ds(base, _BP)], osem)
    g2.wait()
    o2 = pltpu.async_copy(nrows_v, ng_hbm.at[pl.ds(base, _BP)], osem)
    g3.wait()
    o3 = pltpu.async_copy(trows_v, vt_hbm.at[pl.ds(tbase, _BT)], osem)
    o1.wait()
    o2.wait()
    o3.wait()


def _mm_body(x_ref, v_ref, out_ref, sacc_ref, m_ref):
    pid = pl.program_id(0)
    x = x_ref[...]
    v = v_ref[...]
    logits = lax.dot_general(
        x, v, (((1,), (1,)), ((), ())),
        preferred_element_type=jnp.float32)
    if T != 1.0:
        logits = logits * T
    out_ref[...] = logits

    @pl.when(pid == 0)
    def _init():
        nrm0 = jnp.sqrt(jnp.sum(x * x, axis=1, keepdims=True))
        m_ref[...] = nrm0 * T
        sacc_ref[...] = jnp.zeros((B, SACC), jnp.float32)

    colv = lax.broadcasted_iota(jnp.int32, (1, BC), 1) + pid * BC < C
    p = jnp.where(colv, jnp.exp(logits - m_ref[...]), 0.0)
    acc = sacc_ref[...]
    for i in range(BC // SACC):
        acc = acc + p[:, i * SACC:(i + 1) * SACC]
    sacc_ref[...] = acc


@functools.partial(jax.jit, static_argnames=("interpret",))
def _mm_call(inputs, V, interpret=False):
    return pl.pallas_call(
        _mm_body,
        grid=(NB,),
        in_specs=[
            pl.BlockSpec((B, D), lambda i: (0, 0)),
            pl.BlockSpec((BC, D), lambda i: (i, 0)),
        ],
        out_specs=[
            pl.BlockSpec((B, BC), lambda i: (0, i)),
            pl.BlockSpec((B, SACC), lambda i: (0, 0)),
        ],
        out_shape=[
            jax.ShapeDtypeStruct((B, C), jnp.float32),
            jax.ShapeDtypeStruct((B, SACC), jnp.float32),
        ],
        scratch_shapes=[
            pltpu.VMEM((B, 1), jnp.float32),
        ],
        compiler_params=pltpu.CompilerParams(
            dimension_semantics=("arbitrary",)),
        interpret=interpret,
    )(inputs, V)


def _loss_body(ppair_ref, npair_ref, x_ref, sacc_ref, pg_ref, ng_ref, vt_ref,
               loss_ref):
    x = x_ref[...]
    nrm0 = jnp.sqrt(jnp.sum(x * x, axis=1, keepdims=True))
    m = nrm0 * T
    s = jnp.sum(sacc_ref[...], axis=1, keepdims=True)
    lse = m + jnp.log(s)
    tlogit = jnp.sum(x * vt_ref[...], axis=1, keepdims=True) * T
    bu = jnp.sum(lse - tlogit, keepdims=True) / B

    nrm = jnp.maximum(nrm0, 1e-12)
    row = lax.broadcasted_iota(jnp.int32, (B, 1), 0)

    def pair_stats(idx_ref, g_ref):
        vals, valids = [], []
        for p in range(P):
            gcol = idx_ref[:, p:p + 1]
            grow = g_ref[p * B:(p + 1) * B, :]
            d = jnp.sum(x * grow, axis=1, keepdims=True)
            gn = jnp.maximum(
                jnp.sqrt(jnp.sum(grow * grow, axis=1, keepdims=True)), 1e-12)
            val = jnp.clip(d / (nrm * gn), -1.0, 1.0)
            valid = gcol != row
            for q in range(p):
                valid = valid & (gcol != idx_ref[:, q:q + 1])
            vals.append(val)
            valids.append(valid)
        return vals, valids

    pvals, pvalids = pair_stats(ppair_ref, pg_ref)
    nvals, nvalids = pair_stats(npair_ref, ng_ref)

    pmin = jnp.full((B, 1), 2.0, jnp.float32)
    pmax = jnp.full((B, 1), -2.0, jnp.float32)
    for val, valid in zip(pvals, pvalids):
        pmin = jnp.minimum(pmin, jnp.where(valid, val, 2.0))
        pmax = jnp.maximum(pmax, jnp.where(valid, val, -2.0))
    p_thrd = pmax - P_MARGIN
    n_thrd = pmin - N_MARGIN

    def bce_masked(vals, valids, thrd):
        sa = jnp.zeros((B, 1), jnp.float32)
        ca = jnp.zeros((B, 1), jnp.float32)
        for val, valid in zip(vals, valids):
            msk = valid & (val < thrd)
            sa = sa + jnp.where(msk, jnp.log(1.0 + jnp.exp(-val)), 0.0)
            ca = ca + jnp.where(msk, 1.0, 0.0)
        s_tot = jnp.sum(sa, keepdims=True)
        c_tot = jnp.sum(ca, keepdims=True)
        return jnp.where(c_tot > 0, s_tot / jnp.maximum(c_tot, 1.0), 0.0)

    hp_loss = bce_masked(pvals, pvalids, p_thrd)
    hn_loss = bce_masked(nvals, nvalids, n_thrd)
    loss_ref[...] = bu + hp_loss + hn_loss


@functools.partial(jax.jit, static_argnames=("interpret",))
def _loss_call(ppair_idx, npair_idx, inputs, sacc, pg, ng, vt,
               interpret=False):
    return pl.pallas_call(
        _loss_body,
        out_shape=jax.ShapeDtypeStruct((1, 1), jnp.float32),
        interpret=interpret,
    )(ppair_idx, npair_idx, inputs, sacc, pg, ng, vt)


def kernel(inputs, targets, ppair_idx, npair_idx, indexs, V):
    pidx = ppair_idx.T.reshape(-1)
    nidx = npair_idx.T.reshape(-1)
    pg, ng, vt = _sc_gather(inputs, V, pidx, nidx, targets)
    outputs, sacc = _mm_call(inputs, V)
    lossm = _loss_call(ppair_idx, npair_idx, inputs, sacc, pg, ng, vt)
    return lossm[0, 0], outputs


# speedup vs baseline: 1.0813x; 1.0813x over previous
import functools

import jax
import jax.numpy as jnp
from jax import lax
from jax.experimental import pallas as pl
from jax.experimental.pallas import tpu as pltpu
from jax.experimental.pallas import tpu_sc as plsc

B = 1024
D = 128
C = 100000
P = 4
T = 1.0
P_MARGIN = 0.2
N_MARGIN = 0.3

BC = 3072
NB = (C + BC - 1) // BC
SACC = 256

_NC = 2
_NS = 16
_NW = _NC * _NS
_BP = P * B // _NW
_BT = B // _NW


@functools.partial(
    pl.kernel,
    mesh=plsc.VectorSubcoreMesh(core_axis_name="c", subcore_axis_name="s"),
    out_type=[
        jax.ShapeDtypeStruct((P * B, D), jnp.float32),
        jax.ShapeDtypeStruct((P * B, D), jnp.float32),
        jax.ShapeDtypeStruct((B, D), jnp.float32),
    ],
    scratch_types=[
        pltpu.VMEM((_BP,), jnp.int32),
        pltpu.VMEM((_BP,), jnp.int32),
        pltpu.VMEM((_BT,), jnp.int32),
        pltpu.VMEM((_BP, D), jnp.float32),
        pltpu.VMEM((_BP, D), jnp.float32),
        pltpu.VMEM((_BT, D), jnp.float32),
        pltpu.SemaphoreType.DMA,
        pltpu.SemaphoreType.DMA,
    ],
)
def _sc_gather(x_hbm, v_hbm, pidx_hbm, nidx_hbm, tgt_hbm,
               pg_hbm, ng_hbm, vt_hbm,
               pidx_v, nidx_v, tidx_v, prows_v, nrows_v, trows_v,
               sem, osem):
    wid = lax.axis_index("s") * _NC + lax.axis_index("c")
    base = wid * _BP
    tbase = wid * _BT
    i1 = pltpu.async_copy(pidx_hbm.at[pl.ds(base, _BP)], pidx_v, sem)
    i2 = pltpu.async_copy(nidx_hbm.at[pl.ds(base, _BP)], nidx_v, sem)
    i3 = pltpu.async_copy(tgt_hbm.at[pl.ds(tbase, _BT)], tidx_v, sem)
    i1.wait()
    g1 = pltpu.async_copy(x_hbm.at[pidx_v], prows_v, sem)
    i2.wait()
    g2 = pltpu.async_copy(x_hbm.at[nidx_v], nrows_v, sem)
    i3.wait()
    g3 = pltpu.async_copy(v_hbm.at[tidx_v], trows_v, sem)
    g1.wait()
    o1 = pltpu.async_copy(prows_v, pg_hbm.at[pl.ds(base, _BP)], osem)
    g2.wait()
    o2 = pltpu.async_copy(nrows_v, ng_hbm.at[pl.ds(base, _BP)], osem)
    g3.wait()
    o3 = pltpu.async_copy(trows_v, vt_hbm.at[pl.ds(tbase, _BT)], osem)
    o1.wait()
    o2.wait()
    o3.wait()


def _mm_body(x_ref, v_ref, out_ref, sacc_ref, m_ref):
    pid = pl.program_id(0)
    x = x_ref[...]
    v = v_ref[...]
    logits = lax.dot_general(
        x, v, (((1,), (1,)), ((), ())),
        preferred_element_type=jnp.float32)
    if T != 1.0:
        logits = logits * T
    out_ref[...] = logits

    @pl.when(pid == 0)
    def _init():
        nrm0 = jnp.sqrt(jnp.sum(x * x, axis=1, keepdims=True))
        m_ref[...] = nrm0 * T
        sacc_ref[...] = jnp.zeros((B, SACC), jnp.float32)

    colv = lax.broadcasted_iota(jnp.int32, (1, BC), 1) + pid * BC < C
    p = jnp.where(colv, jnp.exp(logits - m_ref[...]), 0.0)
    acc = sacc_ref[...]
    for i in range(BC // SACC):
        acc = acc + p[:, i * SACC:(i + 1) * SACC]
    sacc_ref[...] = acc


@functools.partial(jax.jit, static_argnames=("interpret",))
def _mm_call(inputs, V, interpret=False):
    return pl.pallas_call(
        _mm_body,
        grid=(NB,),
        in_specs=[
            pl.BlockSpec((B, D), lambda i: (0, 0)),
            pl.BlockSpec((BC, D), lambda i: (i, 0)),
        ],
        out_specs=[
            pl.BlockSpec((B, BC), lambda i: (0, i)),
            pl.BlockSpec((B, SACC), lambda i: (0, 0)),
        ],
        out_shape=[
            jax.ShapeDtypeStruct((B, C), jnp.float32),
            jax.ShapeDtypeStruct((B, SACC), jnp.float32),
        ],
        scratch_shapes=[
            pltpu.VMEM((B, 1), jnp.float32),
        ],
        compiler_params=pltpu.CompilerParams(
            dimension_semantics=("arbitrary",)),
        interpret=interpret,
    )(inputs, V)


def _loss_body(ppair_ref, npair_ref, x_ref, sacc_ref, pg_ref, ng_ref, vt_ref,
               loss_ref):
    x = x_ref[...]
    nrm0 = jnp.sqrt(jnp.sum(x * x, axis=1, keepdims=True))
    m = nrm0 * T
    s = jnp.sum(sacc_ref[...], axis=1, keepdims=True)
    lse = m + jnp.log(s)
    tlogit = jnp.sum(x * vt_ref[...], axis=1, keepdims=True) * T
    bu = jnp.sum(lse - tlogit, keepdims=True) / B

    nrm = jnp.maximum(nrm0, 1e-12)
    row = lax.broadcasted_iota(jnp.int32, (B, 1), 0)

    def pair_stats(idx_ref, g_ref):
        vals, valids = [], []
        for p in range(P):
            gcol = idx_ref[:, p:p + 1]
            grow = g_ref[p * B:(p + 1) * B, :]
            d = jnp.sum(x * grow, axis=1, keepdims=True)
            gn = jnp.maximum(
                jnp.sqrt(jnp.sum(grow * grow, axis=1, keepdims=True)), 1e-12)
            val = jnp.clip(d / (nrm * gn), -1.0, 1.0)
            valid = gcol != row
            for q in range(p):
                valid = valid & (gcol != idx_ref[:, q:q + 1])
            vals.append(val)
            valids.append(valid)
        return vals, valids

    pvals, pvalids = pair_stats(ppair_ref, pg_ref)
    nvals, nvalids = pair_stats(npair_ref, ng_ref)

    pmin = jnp.full((B, 1), 2.0, jnp.float32)
    pmax = jnp.full((B, 1), -2.0, jnp.float32)
    for val, valid in zip(pvals, pvalids):
        pmin = jnp.minimum(pmin, jnp.where(valid, val, 2.0))
        pmax = jnp.maximum(pmax, jnp.where(valid, val, -2.0))
    p_thrd = pmax - P_MARGIN
    n_thrd = pmin - N_MARGIN

    def bce_masked(vals, valids, thrd):
        sa = jnp.zeros((B, 1), jnp.float32)
        ca = jnp.zeros((B, 1), jnp.float32)
        for val, valid in zip(vals, valids):
            msk = valid & (val < thrd)
            sa = sa + jnp.where(msk, jnp.log(1.0 + jnp.exp(-val)), 0.0)
            ca = ca + jnp.where(msk, 1.0, 0.0)
        s_tot = jnp.sum(sa, keepdims=True)
        c_tot = jnp.sum(ca, keepdims=True)
        return jnp.where(c_tot > 0, s_tot / jnp.maximum(c_tot, 1.0), 0.0)

    hp_loss = bce_masked(pvals, pvalids, p_thrd)
    hn_loss = bce_masked(nvals, nvalids, n_thrd)
    loss_ref[...] = bu + hp_loss + hn_loss


@functools.partial(jax.jit, static_argnames=("interpret",))
def _loss_call(ppair_idx, npair_idx, inputs, sacc, pg, ng, vt,
               interpret=False):
    return pl.pallas_call(
        _loss_body,
        out_shape=jax.ShapeDtypeStruct((1, 1), jnp.float32),
        interpret=interpret,
    )(ppair_idx, npair_idx, inputs, sacc, pg, ng, vt)


def kernel(inputs, targets, ppair_idx, npair_idx, indexs, V):
    pidx = ppair_idx.T.reshape(-1)
    nidx = npair_idx.T.reshape(-1)
    pg, ng, vt = _sc_gather(inputs, V, pidx, nidx, targets)
    outputs, sacc = _mm_call(inputs, V)
    lossm = _loss_call(ppair_idx, npair_idx, inputs, sacc, pg, ng, vt)
    return lossm[0, 0], outputs
